# baseline (speedup 1.0000x reference)
import functools

import jax
import jax.numpy as jnp
from jax import lax
from jax.experimental import pallas as pl
from jax.experimental.pallas import tpu as pltpu
from jax.experimental.pallas import tpu_sc as plsc

D_MODEL = 128
HEIGHT = 512
WIDTH = 512
N_ROWS = WIDTH * HEIGHT
N = 16384

NUM_CORES = 2
NUM_SUBCORES = 16
NW = NUM_CORES * NUM_SUBCORES
B_PER_W = N // NW
LANES = 16
CHUNK = 64
NCHUNK = B_PER_W // CHUNK


@functools.partial(
    pl.kernel,
    mesh=plsc.VectorSubcoreMesh(core_axis_name="c", subcore_axis_name="s"),
    out_type=jax.ShapeDtypeStruct((N, D_MODEL), jnp.float32),
    scratch_types=[
        pltpu.VMEM((2, B_PER_W), jnp.float32),
        pltpu.VMEM((NCHUNK, CHUNK), jnp.int32),
        pltpu.VMEM((B_PER_W, D_MODEL), jnp.float32),
        pltpu.SemaphoreType.DMA,
    ],
)
def _sc_gather(coords_hbm, table_hbm, out_hbm, cv, idx_v, rows_v, sem):
    wid = lax.axis_index("s") * NUM_CORES + lax.axis_index("c")
    base = wid * B_PER_W
    pltpu.sync_copy(coords_hbm.at[:, pl.ds(base, B_PER_W)], cv)
    for j in range(B_PER_W // LANES):
        xs = cv[0, pl.ds(j * LANES, LANES)]
        ys = cv[1, pl.ds(j * LANES, LANES)]
        xi = ((xs * 1.02 - 0.01) * WIDTH).astype(jnp.int32)
        yi = ((ys * 1.02 - 0.01) * HEIGHT).astype(jnp.int32)
        xi = jnp.minimum(jnp.maximum(xi, 0), WIDTH)
        yi = jnp.minimum(jnp.maximum(yi, 0), HEIGHT)
        idx = jnp.minimum(xi * WIDTH + yi, N_ROWS - 1)
        idx_v[(j * LANES) // CHUNK, pl.ds((j * LANES) % CHUNK, LANES)] = idx
    gathers = [
        pltpu.async_copy(
            table_hbm.at[idx_v.at[c]], rows_v.at[pl.ds(c * CHUNK, CHUNK)], sem
        )
        for c in range(NCHUNK)
    ]
    for cp in gathers:
        cp.wait()
    pltpu.sync_copy(rows_v, out_hbm.at[pl.ds(base, B_PER_W)])


def kernel(coordinates, pe_table, missing_pe):
    is_missing = coordinates[0, 0] == -1
    coords_t = coordinates.T
    return lax.cond(
        is_missing,
        lambda: jnp.broadcast_to(missing_pe[None, :], (N, D_MODEL)),
        lambda: _sc_gather(coords_t, pe_table),
    )


# speedup vs baseline: 1.0108x; 1.0108x over previous
import functools

import jax
import jax.numpy as jnp
from jax import lax
from jax.experimental import pallas as pl
from jax.experimental.pallas import tpu as pltpu
from jax.experimental.pallas import tpu_sc as plsc

D_MODEL = 128
HEIGHT = 512
WIDTH = 512
N_ROWS = WIDTH * HEIGHT
N = 16384

NUM_CORES = 2
NUM_SUBCORES = 16
NW = NUM_CORES * NUM_SUBCORES
B_PER_W = N // NW
LANES = 16
CHUNK = 32
NCHUNK = B_PER_W // CHUNK


@functools.partial(
    pl.kernel,
    mesh=plsc.VectorSubcoreMesh(core_axis_name="c", subcore_axis_name="s"),
    out_type=jax.ShapeDtypeStruct((N, D_MODEL), jnp.float32),
    scratch_types=[
        pltpu.VMEM((2, B_PER_W), jnp.float32),
        pltpu.VMEM((NCHUNK, CHUNK), jnp.int32),
        pltpu.VMEM((B_PER_W, D_MODEL), jnp.float32),
        pltpu.SemaphoreType.DMA,
    ],
)
def _sc_gather(coords_hbm, table_hbm, out_hbm, cv, idx_v, rows_v, sem):
    wid = lax.axis_index("s") * NUM_CORES + lax.axis_index("c")
    base = wid * B_PER_W
    pltpu.sync_copy(coords_hbm.at[:, pl.ds(base, B_PER_W)], cv)
    for j in range(B_PER_W // LANES):
        xs = cv[0, pl.ds(j * LANES, LANES)]
        ys = cv[1, pl.ds(j * LANES, LANES)]
        xi = ((xs * 1.02 - 0.01) * WIDTH).astype(jnp.int32)
        yi = ((ys * 1.02 - 0.01) * HEIGHT).astype(jnp.int32)
        xi = jnp.minimum(jnp.maximum(xi, 0), WIDTH)
        yi = jnp.minimum(jnp.maximum(yi, 0), HEIGHT)
        idx = jnp.minimum(xi * WIDTH + yi, N_ROWS - 1)
        idx_v[(j * LANES) // CHUNK, pl.ds((j * LANES) % CHUNK, LANES)] = idx
    gathers = [
        pltpu.async_copy(
            table_hbm.at[idx_v.at[c]], rows_v.at[pl.ds(c * CHUNK, CHUNK)], sem
        )
        for c in range(NCHUNK)
    ]
    for cp in gathers:
        cp.wait()
    pltpu.sync_copy(rows_v, out_hbm.at[pl.ds(base, B_PER_W)])


def kernel(coordinates, pe_table, missing_pe):
    is_missing = coordinates[0, 0] == -1
    coords_t = coordinates.T
    return lax.cond(
        is_missing,
        lambda: jnp.broadcast_to(missing_pe[None, :], (N, D_MODEL)),
        lambda: _sc_gather(coords_t, pe_table),
    )


# speedup vs baseline: 1.0140x; 1.0032x over previous
import functools

import jax
import jax.numpy as jnp
from jax import lax
from jax.experimental import pallas as pl
from jax.experimental.pallas import tpu as pltpu
from jax.experimental.pallas import tpu_sc as plsc

D_MODEL = 128
HEIGHT = 512
WIDTH = 512
N_ROWS = WIDTH * HEIGHT
N = 16384

NUM_CORES = 2
NUM_SUBCORES = 16
NW = NUM_CORES * NUM_SUBCORES
B_PER_W = N // NW
LANES = 16
CHUNK = 128
NCHUNK = B_PER_W // CHUNK


@functools.partial(
    pl.kernel,
    mesh=plsc.VectorSubcoreMesh(core_axis_name="c", subcore_axis_name="s"),
    out_type=jax.ShapeDtypeStruct((N, D_MODEL), jnp.float32),
    scratch_types=[
        pltpu.VMEM((2, B_PER_W), jnp.float32),
        pltpu.VMEM((NCHUNK, CHUNK), jnp.int32),
        pltpu.VMEM((B_PER_W, D_MODEL), jnp.float32),
        pltpu.SemaphoreType.DMA,
    ],
)
def _sc_gather(coords_hbm, table_hbm, out_hbm, cv, idx_v, rows_v, sem):
    wid = lax.axis_index("s") * NUM_CORES + lax.axis_index("c")
    base = wid * B_PER_W
    pltpu.sync_copy(coords_hbm.at[:, pl.ds(base, B_PER_W)], cv)
    for j in range(B_PER_W // LANES):
        xs = cv[0, pl.ds(j * LANES, LANES)]
        ys = cv[1, pl.ds(j * LANES, LANES)]
        xi = ((xs * 1.02 - 0.01) * WIDTH).astype(jnp.int32)
        yi = ((ys * 1.02 - 0.01) * HEIGHT).astype(jnp.int32)
        xi = jnp.minimum(jnp.maximum(xi, 0), WIDTH)
        yi = jnp.minimum(jnp.maximum(yi, 0), HEIGHT)
        idx = jnp.minimum(xi * WIDTH + yi, N_ROWS - 1)
        idx_v[(j * LANES) // CHUNK, pl.ds((j * LANES) % CHUNK, LANES)] = idx
    for c in range(NCHUNK):
        pltpu.async_copy(
            table_hbm.at[idx_v.at[c]], rows_v.at[pl.ds(c * CHUNK, CHUNK)], sem
        )
    pltpu.make_async_copy(table_hbm.at[pl.ds(0, B_PER_W)], rows_v, sem).wait()
    pltpu.sync_copy(rows_v, out_hbm.at[pl.ds(base, B_PER_W)])


def kernel(coordinates, pe_table, missing_pe):
    is_missing = coordinates[0, 0] == -1
    coords_t = coordinates.T
    return lax.cond(
        is_missing,
        lambda: jnp.broadcast_to(missing_pe[None, :], (N, D_MODEL)),
        lambda: _sc_gather(coords_t, pe_table),
    )
